# baseline (speedup 1.0000x reference)
import functools

import jax
import jax.numpy as jnp
from jax import lax
from jax.experimental import pallas as pl
from jax.experimental.pallas import tpu as pltpu

_NUM_GROUP = 128
_GROUP_SIZE = 32
_GRID_PTS = 20
_PERCENT = 0.1
_SEL = 800
_SEL_PAD = 1024


def _fps_jax(points, n_samples):
    def body(carry, _):
        dist, last = carry
        d = jnp.sum((points - points[last]) ** 2, axis=1)
        dist = jnp.minimum(dist, d)
        nxt = jnp.argmax(dist).astype(jnp.int32)
        return (dist, nxt), nxt

    init = (jnp.full((points.shape[0],), 1e10, dtype=points.dtype),
            jnp.array(0, dtype=jnp.int32))
    _, rest = jax.lax.scan(body, init, None, length=n_samples - 1, unroll=32)
    idx = jnp.concatenate([jnp.zeros((1,), jnp.int32), rest.astype(jnp.int32)])
    return points[idx]


def _knn_kernel(xt_ref, xyz_ref, cen_ref, out_ref, d_ref):
    xr0 = xt_ref[0, 0:1, :]
    xr1 = xt_ref[0, 1:2, :]
    xr2 = xt_ref[0, 2:3, :]
    cx = cen_ref[:, 0:1]
    cy = cen_ref[:, 1:2]
    cz = cen_ref[:, 2:3]
    d_ref[...] = (cx - xr0) ** 2 + (cy - xr1) ** 2 + (cz - xr2) ** 2

    xyzb = xyz_ref[0]
    cen = cen_ref[...]
    il = lax.broadcasted_iota(jnp.int32, (128, 2048), 1)

    hi = xyzb.astype(jnp.bfloat16)
    mid = (xyzb - hi.astype(jnp.float32)).astype(jnp.bfloat16)
    lo = (xyzb - hi.astype(jnp.float32)
          - mid.astype(jnp.float32)).astype(jnp.bfloat16)
    ones = jnp.ones((xyzb.shape[0], 1), jnp.bfloat16)
    hml = jnp.concatenate([hi, mid, lo, ones], axis=1)

    for k in range(_GROUP_SIZE):
        D = d_ref[...]
        m = jnp.min(D, axis=1, keepdims=True)
        eq = D == m
        ef = eq.astype(jnp.bfloat16)
        c10 = jnp.dot(ef, hml, preferred_element_type=jnp.float32)
        coords = (c10[:, 0:3] + c10[:, 3:6]) + c10[:, 6:9]
        out_ref[0, k, :, :] = coords - cen
        d_ref[...] = jnp.where(eq, jnp.float32(jnp.inf), D)

        @pl.when(jnp.max(c10[:, 9]) > 1.5)
        def _tie_fixup():
            cand = jnp.where(eq, il, jnp.int32(4096))
            j = jnp.min(cand, axis=1, keepdims=True)
            pick = il == j
            pf = pick.astype(jnp.bfloat16)
            c9p = jnp.dot(pf, hml, preferred_element_type=jnp.float32)
            pcoords = (c9p[:, 0:3] + c9p[:, 3:6]) + c9p[:, 6:9]
            out_ref[0, k, :, :] = pcoords - cen
            d_ref[...] = jnp.where(pick, jnp.float32(jnp.inf), D)


def _adjust_range(min_value, max_value, n):
    adjusted_min = min_value + (max_value - min_value) / (n + 1)
    adjusted_max = max_value - (max_value - min_value) / (n + 1)
    return adjusted_min, adjusted_max


@jax.jit
def kernel(xyz):
    B, N, _ = xyz.shape
    pts = xyz.reshape(-1, 3)

    min_coords = pts.min(axis=0)
    max_coords = pts.max(axis=0)
    x_min, x_max = _adjust_range(min_coords[0], max_coords[0], _GRID_PTS)
    y_min, y_max = _adjust_range(min_coords[1], max_coords[1], _GRID_PTS)
    z_min, z_max = _adjust_range(min_coords[2], max_coords[2], _GRID_PTS)
    x_points = jnp.linspace(x_min, x_max, _GRID_PTS)
    y_points = jnp.linspace(y_min, y_max, _GRID_PTS)
    z_points = jnp.linspace(z_min, z_max, _GRID_PTS)
    X, Y, Z = jnp.meshgrid(x_points, y_points, z_points, indexing='ij')
    centers = jnp.stack([X, Y, Z], axis=-1).reshape(-1, 3)

    sq_p = jnp.sum(pts ** 2)
    S = jnp.sum(pts, axis=0)
    Np = pts.shape[0]
    total = sq_p - 2.0 * centers @ S + Np * jnp.sum(centers ** 2, axis=1)
    _, order = jax.lax.top_k(-total, _SEL)
    sel = centers[order]

    fps_centers = _fps_jax(sel, _NUM_GROUP)

    xt = xyz.transpose(0, 2, 1)
    neigh_t = pl.pallas_call(
        _knn_kernel,
        grid=(B,),
        in_specs=[
            pl.BlockSpec((1, 3, N), lambda b: (b, 0, 0)),
            pl.BlockSpec((1, N, 3), lambda b: (b, 0, 0)),
            pl.BlockSpec((_NUM_GROUP, 3), lambda b: (0, 0)),
        ],
        out_specs=pl.BlockSpec((1, _GROUP_SIZE, _NUM_GROUP, 3),
                               lambda b: (b, 0, 0, 0)),
        out_shape=jax.ShapeDtypeStruct((B, _GROUP_SIZE, _NUM_GROUP, 3),
                                       jnp.float32),
        scratch_shapes=[pltpu.VMEM((_NUM_GROUP, N), jnp.float32)],
    )(xt, xyz, fps_centers)

    neighborhood = neigh_t.transpose(0, 2, 1, 3)
    center = jnp.broadcast_to(fps_centers[None], (B, _NUM_GROUP, 3))
    return (neighborhood, center)


# speedup vs baseline: 1.6940x; 1.6940x over previous
import functools

import jax
import jax.numpy as jnp
from jax import lax
from jax.experimental import pallas as pl
from jax.experimental.pallas import tpu as pltpu

_NUM_GROUP = 128
_GROUP_SIZE = 32
_GRID_PTS = 20
_PERCENT = 0.1
_SEL = 800
_SEL_PAD = 1024


def _fps_jax(points, n_samples):
    def body(carry, _):
        dist, last = carry
        d = jnp.sum((points - points[last]) ** 2, axis=1)
        dist = jnp.minimum(dist, d)
        nxt = jnp.argmax(dist).astype(jnp.int32)
        return (dist, nxt), nxt

    init = (jnp.full((points.shape[0],), 1e10, dtype=points.dtype),
            jnp.array(0, dtype=jnp.int32))
    _, rest = jax.lax.scan(body, init, None, length=n_samples - 1, unroll=32)
    idx = jnp.concatenate([jnp.zeros((1,), jnp.int32), rest.astype(jnp.int32)])
    return points[idx]


def _knn_kernel(xt_ref, xyz_ref, cen_ref, out_ref):
    xr0 = xt_ref[0, 0:1, :]
    xr1 = xt_ref[0, 1:2, :]
    xr2 = xt_ref[0, 2:3, :]
    cx = cen_ref[:, 0:1]
    cy = cen_ref[:, 1:2]
    cz = cen_ref[:, 2:3]
    D = (cx - xr0) ** 2 + (cy - xr1) ** 2 + (cz - xr2) ** 2

    xyzb = xyz_ref[0]
    cen = cen_ref[...]
    ilf = lax.broadcasted_iota(jnp.int32, (128, 2048), 1).astype(jnp.float32)

    hi = xyzb.astype(jnp.bfloat16)
    mid = (xyzb - hi.astype(jnp.float32)).astype(jnp.bfloat16)
    lo = (xyzb - hi.astype(jnp.float32)
          - mid.astype(jnp.float32)).astype(jnp.bfloat16)
    ones = jnp.ones((xyzb.shape[0], 1), jnp.bfloat16)
    hml = jnp.concatenate([hi, mid, lo, ones], axis=1)

    for k in range(_GROUP_SIZE):
        m = jnp.min(D, axis=1, keepdims=True)
        eq = D == m
        cand = jnp.where(eq, ilf, jnp.float32(4096.0))
        j = jnp.min(cand, axis=1, keepdims=True)
        pick = cand == j
        pf = pick.astype(jnp.bfloat16)
        c9 = jnp.dot(pf, hml, preferred_element_type=jnp.float32)
        coords = (c9[:, 0:3] + c9[:, 3:6]) + c9[:, 6:9]
        out_ref[0, k, :, :] = coords - cen
        D = jnp.where(pick, jnp.float32(jnp.inf), D)


def _adjust_range(min_value, max_value, n):
    adjusted_min = min_value + (max_value - min_value) / (n + 1)
    adjusted_max = max_value - (max_value - min_value) / (n + 1)
    return adjusted_min, adjusted_max


@jax.jit
def kernel(xyz):
    B, N, _ = xyz.shape
    pts = xyz.reshape(-1, 3)

    min_coords = pts.min(axis=0)
    max_coords = pts.max(axis=0)
    x_min, x_max = _adjust_range(min_coords[0], max_coords[0], _GRID_PTS)
    y_min, y_max = _adjust_range(min_coords[1], max_coords[1], _GRID_PTS)
    z_min, z_max = _adjust_range(min_coords[2], max_coords[2], _GRID_PTS)
    x_points = jnp.linspace(x_min, x_max, _GRID_PTS)
    y_points = jnp.linspace(y_min, y_max, _GRID_PTS)
    z_points = jnp.linspace(z_min, z_max, _GRID_PTS)
    X, Y, Z = jnp.meshgrid(x_points, y_points, z_points, indexing='ij')
    centers = jnp.stack([X, Y, Z], axis=-1).reshape(-1, 3)

    sq_p = jnp.sum(pts ** 2)
    S = jnp.sum(pts, axis=0)
    Np = pts.shape[0]
    total = sq_p - 2.0 * centers @ S + Np * jnp.sum(centers ** 2, axis=1)
    _, order = jax.lax.top_k(-total, _SEL)
    sel = centers[order]

    fps_centers = _fps_jax(sel, _NUM_GROUP)

    xt = xyz.transpose(0, 2, 1)
    neigh_t = pl.pallas_call(
        _knn_kernel,
        grid=(B,),
        in_specs=[
            pl.BlockSpec((1, 3, N), lambda b: (b, 0, 0)),
            pl.BlockSpec((1, N, 3), lambda b: (b, 0, 0)),
            pl.BlockSpec((_NUM_GROUP, 3), lambda b: (0, 0)),
        ],
        out_specs=pl.BlockSpec((1, _GROUP_SIZE, _NUM_GROUP, 3),
                               lambda b: (b, 0, 0, 0)),
        out_shape=jax.ShapeDtypeStruct((B, _GROUP_SIZE, _NUM_GROUP, 3),
                                       jnp.float32),
    )(xt, xyz, fps_centers)

    neighborhood = neigh_t.transpose(0, 2, 1, 3)
    center = jnp.broadcast_to(fps_centers[None], (B, _NUM_GROUP, 3))
    return (neighborhood, center)


# speedup vs baseline: 2.0154x; 1.1897x over previous
import functools

import jax
import jax.numpy as jnp
from jax import lax
from jax.experimental import pallas as pl
from jax.experimental.pallas import tpu as pltpu

_NUM_GROUP = 128
_GROUP_SIZE = 32
_GRID_PTS = 20
_PERCENT = 0.1
_SEL = 800
_SEL_PAD = 1024


def _fps_jax(points, n_samples):
    def body(carry, _):
        dist, last = carry
        d = jnp.sum((points - points[last]) ** 2, axis=1)
        dist = jnp.minimum(dist, d)
        nxt = jnp.argmax(dist).astype(jnp.int32)
        return (dist, nxt), nxt

    init = (jnp.full((points.shape[0],), 1e10, dtype=points.dtype),
            jnp.array(0, dtype=jnp.int32))
    _, rest = jax.lax.scan(body, init, None, length=n_samples - 1, unroll=32)
    idx = jnp.concatenate([jnp.zeros((1,), jnp.int32), rest.astype(jnp.int32)])
    return points[idx]


def _knn_kernel(xt_ref, xyz_ref, cen_ref, out_ref):
    xr0 = xt_ref[0, 0:1, :]
    xr1 = xt_ref[0, 1:2, :]
    xr2 = xt_ref[0, 2:3, :]
    cx = cen_ref[:, 0:1]
    cy = cen_ref[:, 1:2]
    cz = cen_ref[:, 2:3]
    D = (cx - xr0) ** 2 + (cy - xr1) ** 2 + (cz - xr2) ** 2

    xyzb = xyz_ref[0]
    cen = cen_ref[...]
    ilf = lax.broadcasted_iota(jnp.int32, (128, 2048), 1).astype(jnp.float32)

    hi = xyzb.astype(jnp.bfloat16)
    mid = (xyzb - hi.astype(jnp.float32)).astype(jnp.bfloat16)
    lo = (xyzb - hi.astype(jnp.float32)
          - mid.astype(jnp.float32)).astype(jnp.bfloat16)
    ones = jnp.ones((xyzb.shape[0], 1), jnp.bfloat16)
    hml = jnp.concatenate([hi, mid, lo, ones], axis=1)

    D0 = D
    tie = jnp.float32(0.0)
    for k in range(_GROUP_SIZE):
        m = jnp.min(D, axis=1, keepdims=True)
        eq = D == m
        ef = eq.astype(jnp.bfloat16)
        c9 = jnp.dot(ef, hml, preferred_element_type=jnp.float32)
        tie = jnp.maximum(tie, jnp.max(c9[:, 9]))
        coords = (c9[:, 0:3] + c9[:, 3:6]) + c9[:, 6:9]
        out_ref[0, k, :, :] = coords - cen
        D = jnp.where(eq, jnp.float32(jnp.inf), D)

    @pl.when(tie > 1.5)
    def _precise_redo():
        Dp = D0
        for k in range(_GROUP_SIZE):
            m = jnp.min(Dp, axis=1, keepdims=True)
            eq = Dp == m
            cand = jnp.where(eq, ilf, jnp.float32(4096.0))
            j = jnp.min(cand, axis=1, keepdims=True)
            pick = cand == j
            pf = pick.astype(jnp.bfloat16)
            c9 = jnp.dot(pf, hml, preferred_element_type=jnp.float32)
            coords = (c9[:, 0:3] + c9[:, 3:6]) + c9[:, 6:9]
            out_ref[0, k, :, :] = coords - cen
            Dp = jnp.where(pick, jnp.float32(jnp.inf), Dp)


def _adjust_range(min_value, max_value, n):
    adjusted_min = min_value + (max_value - min_value) / (n + 1)
    adjusted_max = max_value - (max_value - min_value) / (n + 1)
    return adjusted_min, adjusted_max


@jax.jit
def kernel(xyz):
    B, N, _ = xyz.shape
    pts = xyz.reshape(-1, 3)

    min_coords = pts.min(axis=0)
    max_coords = pts.max(axis=0)
    x_min, x_max = _adjust_range(min_coords[0], max_coords[0], _GRID_PTS)
    y_min, y_max = _adjust_range(min_coords[1], max_coords[1], _GRID_PTS)
    z_min, z_max = _adjust_range(min_coords[2], max_coords[2], _GRID_PTS)
    x_points = jnp.linspace(x_min, x_max, _GRID_PTS)
    y_points = jnp.linspace(y_min, y_max, _GRID_PTS)
    z_points = jnp.linspace(z_min, z_max, _GRID_PTS)
    X, Y, Z = jnp.meshgrid(x_points, y_points, z_points, indexing='ij')
    centers = jnp.stack([X, Y, Z], axis=-1).reshape(-1, 3)

    sq_p = jnp.sum(pts ** 2)
    S = jnp.sum(pts, axis=0)
    Np = pts.shape[0]
    total = sq_p - 2.0 * centers @ S + Np * jnp.sum(centers ** 2, axis=1)
    _, order = jax.lax.top_k(-total, _SEL)
    sel = centers[order]

    fps_centers = _fps_jax(sel, _NUM_GROUP)

    xt = xyz.transpose(0, 2, 1)
    neigh_t = pl.pallas_call(
        _knn_kernel,
        grid=(B,),
        in_specs=[
            pl.BlockSpec((1, 3, N), lambda b: (b, 0, 0)),
            pl.BlockSpec((1, N, 3), lambda b: (b, 0, 0)),
            pl.BlockSpec((_NUM_GROUP, 3), lambda b: (0, 0)),
        ],
        out_specs=pl.BlockSpec((1, _GROUP_SIZE, _NUM_GROUP, 3),
                               lambda b: (b, 0, 0, 0)),
        out_shape=jax.ShapeDtypeStruct((B, _GROUP_SIZE, _NUM_GROUP, 3),
                                       jnp.float32),
    )(xt, xyz, fps_centers)

    neighborhood = neigh_t.transpose(0, 2, 1, 3)
    center = jnp.broadcast_to(fps_centers[None], (B, _NUM_GROUP, 3))
    return (neighborhood, center)


# speedup vs baseline: 2.0155x; 1.0000x over previous
import jax
import jax.numpy as jnp
from jax import lax
from jax.experimental import pallas as pl

_NUM_GROUP = 128
_GROUP_SIZE = 32
_GRID_PTS = 20
_SEL = 800


def _fps_jax(points, n_samples):
    def body(carry, _):
        dist, last = carry
        d = jnp.sum((points - points[last]) ** 2, axis=1)
        dist = jnp.minimum(dist, d)
        nxt = jnp.argmax(dist).astype(jnp.int32)
        return (dist, nxt), nxt

    init = (jnp.full((points.shape[0],), 1e10, dtype=points.dtype),
            jnp.array(0, dtype=jnp.int32))
    _, rest = jax.lax.scan(body, init, None, length=n_samples - 1, unroll=32)
    idx = jnp.concatenate([jnp.zeros((1,), jnp.int32), rest.astype(jnp.int32)])
    return points[idx]


def _knn_kernel(xt_ref, xyz_ref, cen_ref, out_ref):
    xr0 = xt_ref[0, 0:1, :]
    xr1 = xt_ref[0, 1:2, :]
    xr2 = xt_ref[0, 2:3, :]
    cx = cen_ref[:, 0:1]
    cy = cen_ref[:, 1:2]
    cz = cen_ref[:, 2:3]
    D = (cx - xr0) ** 2 + (cy - xr1) ** 2 + (cz - xr2) ** 2

    xyzb = xyz_ref[0]
    cen = cen_ref[...]
    ilf = lax.broadcasted_iota(jnp.int32, (128, 2048), 1).astype(jnp.float32)

    hi = xyzb.astype(jnp.bfloat16)
    mid = (xyzb - hi.astype(jnp.float32)).astype(jnp.bfloat16)
    lo = (xyzb - hi.astype(jnp.float32)
          - mid.astype(jnp.float32)).astype(jnp.bfloat16)
    ones = jnp.ones((xyzb.shape[0], 1), jnp.bfloat16)
    hml = jnp.concatenate([hi, mid, lo, ones], axis=1)

    D0 = D
    tie = jnp.float32(0.0)
    for k in range(_GROUP_SIZE):
        m = jnp.min(D, axis=1, keepdims=True)
        eq = D == m
        ef = eq.astype(jnp.bfloat16)
        c9 = jnp.dot(ef, hml, preferred_element_type=jnp.float32)
        tie = jnp.maximum(tie, jnp.max(c9[:, 9]))
        coords = (c9[:, 0:3] + c9[:, 3:6]) + c9[:, 6:9]
        out_ref[0, k, :, :] = coords - cen
        D = jnp.where(eq, jnp.float32(jnp.inf), D)

    @pl.when(tie > 1.5)
    def _precise_redo():
        Dp = D0
        for k in range(_GROUP_SIZE):
            m = jnp.min(Dp, axis=1, keepdims=True)
            eq = Dp == m
            cand = jnp.where(eq, ilf, jnp.float32(4096.0))
            j = jnp.min(cand, axis=1, keepdims=True)
            pick = cand == j
            pf = pick.astype(jnp.bfloat16)
            c9 = jnp.dot(pf, hml, preferred_element_type=jnp.float32)
            coords = (c9[:, 0:3] + c9[:, 3:6]) + c9[:, 6:9]
            out_ref[0, k, :, :] = coords - cen
            Dp = jnp.where(pick, jnp.float32(jnp.inf), Dp)


def _adjust_range(min_value, max_value, n):
    adjusted_min = min_value + (max_value - min_value) / (n + 1)
    adjusted_max = max_value - (max_value - min_value) / (n + 1)
    return adjusted_min, adjusted_max


@jax.jit
def kernel(xyz):
    B, N, _ = xyz.shape
    pts = xyz.reshape(-1, 3)

    min_coords = pts.min(axis=0)
    max_coords = pts.max(axis=0)
    x_min, x_max = _adjust_range(min_coords[0], max_coords[0], _GRID_PTS)
    y_min, y_max = _adjust_range(min_coords[1], max_coords[1], _GRID_PTS)
    z_min, z_max = _adjust_range(min_coords[2], max_coords[2], _GRID_PTS)
    x_points = jnp.linspace(x_min, x_max, _GRID_PTS)
    y_points = jnp.linspace(y_min, y_max, _GRID_PTS)
    z_points = jnp.linspace(z_min, z_max, _GRID_PTS)
    X, Y, Z = jnp.meshgrid(x_points, y_points, z_points, indexing='ij')
    centers = jnp.stack([X, Y, Z], axis=-1).reshape(-1, 3)

    sq_p = jnp.sum(pts ** 2)
    S = jnp.sum(pts, axis=0)
    Np = pts.shape[0]
    total = sq_p - 2.0 * centers @ S + Np * jnp.sum(centers ** 2, axis=1)
    _, order = jax.lax.top_k(-total, _SEL)
    sel = centers[order]

    fps_centers = _fps_jax(sel, _NUM_GROUP)

    xt = xyz.transpose(0, 2, 1)
    neigh_t = pl.pallas_call(
        _knn_kernel,
        grid=(B,),
        in_specs=[
            pl.BlockSpec((1, 3, N), lambda b: (b, 0, 0)),
            pl.BlockSpec((1, N, 3), lambda b: (b, 0, 0)),
            pl.BlockSpec((_NUM_GROUP, 3), lambda b: (0, 0)),
        ],
        out_specs=pl.BlockSpec((1, _GROUP_SIZE, _NUM_GROUP, 3),
                               lambda b: (b, 0, 0, 0)),
        out_shape=jax.ShapeDtypeStruct((B, _GROUP_SIZE, _NUM_GROUP, 3),
                                       jnp.float32),
    )(xt, xyz, fps_centers)

    neighborhood = neigh_t.transpose(0, 2, 1, 3)
    center = jnp.broadcast_to(fps_centers[None], (B, _NUM_GROUP, 3))
    return (neighborhood, center)
